# baseline (speedup 1.0000x reference)
import functools

import jax
import jax.numpy as jnp
from jax import lax
from jax.experimental import pallas as pl
from jax.experimental.pallas import tpu as pltpu
from jax.experimental.pallas import tpu_sc as plsc

BATCH = 4096
SEQ_LEN = 200
EMBED_DIM = 64
TOTAL = BATCH * SEQ_LEN

NUM_CORES = 2
NUM_SUBCORES = 16
NUM_WORKERS = NUM_CORES * NUM_SUBCORES
PER_WORKER = TOTAL // NUM_WORKERS

CHUNK = 2 * SEQ_LEN
NUM_CHUNKS = PER_WORKER // CHUNK
IDX_W = 100
IDX_ROWS = CHUNK // IDX_W
IDX_ALL = PER_WORKER // IDX_W


def kernel(inputs, pos_table, tok_table):
    idx2d = inputs.reshape(TOTAL // IDX_W, IDX_W).astype(jnp.int32)

    mesh = plsc.VectorSubcoreMesh(core_axis_name="c", subcore_axis_name="s")

    @functools.partial(
        pl.kernel,
        out_type=jax.ShapeDtypeStruct((TOTAL, EMBED_DIM), jnp.float32),
        mesh=mesh,
        scratch_types=[
            pltpu.VMEM((IDX_ALL, IDX_W), jnp.int32),
            pltpu.VMEM((CHUNK, EMBED_DIM), jnp.float32),
            pltpu.VMEM((CHUNK, EMBED_DIM), jnp.float32),
            pltpu.SemaphoreType.DMA,
            pltpu.SemaphoreType.DMA,
            pltpu.SemaphoreType.DMA,
            pltpu.SemaphoreType.DMA,
            pltpu.SemaphoreType.DMA,
            pltpu.SemaphoreType.DMA,
        ],
        compiler_params=pltpu.CompilerParams(use_tc_tiling_on_sc=False),
    )
    def embed(idx_hbm, pos_hbm, tok_hbm, out_hbm,
              idx_v, rows0, rows1,
              psem0, psem1, gsem0, gsem1, ssem0, ssem1):
        rows = (rows0, rows1)
        psem = (psem0, psem1)
        gsem = (gsem0, gsem1)
        ssem = (ssem0, ssem1)

        wid = lax.axis_index("s") * NUM_CORES + lax.axis_index("c")
        row_base = wid * PER_WORKER
        idx_base = wid * IDX_ALL

        pltpu.sync_copy(idx_hbm.at[pl.ds(idx_base, IDX_ALL)], idx_v)

        def start_fill(b):
            pltpu.async_copy(pos_hbm, rows[b].at[pl.ds(0, SEQ_LEN)], psem[b])
            pltpu.async_copy(pos_hbm, rows[b].at[pl.ds(SEQ_LEN, SEQ_LEN)],
                             psem[b])

        def wait_fill(b):
            for r in range(2):
                pltpu.make_async_copy(pos_hbm,
                                      rows[b].at[pl.ds(r * SEQ_LEN, SEQ_LEN)],
                                      psem[b]).wait()

        def start_gathers(c, b):
            for j in range(IDX_ROWS):
                pltpu.async_copy(tok_hbm.at[idx_v.at[c * IDX_ROWS + j]],
                                 rows[b].at[pl.ds(j * IDX_W, IDX_W)], gsem[b],
                                 add=True)

        def wait_gathers(b):
            for j in range(IDX_ROWS):
                pltpu.make_async_copy(tok_hbm.at[idx_v.at[j]],
                                      rows[b].at[pl.ds(j * IDX_W, IDX_W)],
                                      gsem[b]).wait()

        def start_store(c, b):
            pltpu.async_copy(rows[b],
                             out_hbm.at[pl.ds(row_base + c * CHUNK, CHUNK)],
                             ssem[b])

        def wait_store(b):
            pltpu.make_async_copy(rows[b],
                                  out_hbm.at[pl.ds(row_base, CHUNK)],
                                  ssem[b]).wait()

        for b in range(2):
            start_fill(b)
            wait_fill(b)
            start_gathers(b, b)

        @pl.loop(0, NUM_CHUNKS, step=2)
        def _(cc):
            for b in range(2):
                c = cc + b
                wait_gathers(b)
                start_store(c, b)
                @pl.when(c + 2 < NUM_CHUNKS)
                def _():
                    wait_store(b)
                    start_fill(b)
                    wait_fill(b)
                    start_gathers(c + 2, b)

        wait_store(0)
        wait_store(1)

    out = embed(idx2d, pos_table, tok_table)
    return out.reshape(BATCH, SEQ_LEN, EMBED_DIM)


# speedup vs baseline: 1.7246x; 1.7246x over previous
import functools

import jax
import jax.numpy as jnp
from jax import lax
from jax.experimental import pallas as pl
from jax.experimental.pallas import tpu as pltpu
from jax.experimental.pallas import tpu_sc as plsc

BATCH = 4096
SEQ_LEN = 200
EMBED_DIM = 64
TOTAL = BATCH * SEQ_LEN

NUM_CORES = 2
NUM_SUBCORES = 16
NUM_WORKERS = NUM_CORES * NUM_SUBCORES
PER_WORKER = TOTAL // NUM_WORKERS

CHUNK = SEQ_LEN
NUM_CHUNKS = PER_WORKER // CHUNK
IDX_W = 100
IDX_ROWS = CHUNK // IDX_W
IDX_ALL = PER_WORKER // IDX_W
NBUF = 4


def kernel(inputs, pos_table, tok_table):
    idx2d = inputs.reshape(TOTAL // IDX_W, IDX_W).astype(jnp.int32)
    posidx = jnp.arange(CHUNK, dtype=jnp.int32).reshape(IDX_ROWS, IDX_W)

    mesh = plsc.VectorSubcoreMesh(core_axis_name="c", subcore_axis_name="s")

    @functools.partial(
        pl.kernel,
        out_type=jax.ShapeDtypeStruct((TOTAL, EMBED_DIM), jnp.float32),
        mesh=mesh,
        scratch_types=[
            pltpu.VMEM((IDX_ALL, IDX_W), jnp.int32),
            pltpu.VMEM((IDX_ROWS, IDX_W), jnp.int32),
            [pltpu.VMEM((CHUNK, EMBED_DIM), jnp.float32)
             for _ in range(NBUF)],
            pltpu.VMEM_SHARED((SEQ_LEN, EMBED_DIM), jnp.float32),
            [pltpu.SemaphoreType.DMA for _ in range(NBUF)],
            [pltpu.SemaphoreType.DMA for _ in range(NBUF)],
            [pltpu.SemaphoreType.DMA for _ in range(NBUF)],
        ],
        compiler_params=pltpu.CompilerParams(use_tc_tiling_on_sc=False),
    )
    def embed(idx_hbm, posidx_hbm, pos_hbm, tok_hbm, out_hbm,
              idx_v, posidx_v, rows, pos_sh, gsem, psem, ssem):
        wid = lax.axis_index("s") * NUM_CORES + lax.axis_index("c")
        row_base = wid * PER_WORKER
        idx_base = wid * IDX_ALL

        @pl.when(lax.axis_index("s") == 0)
        def _():
            pltpu.sync_copy(pos_hbm, rows[0])
            pltpu.sync_copy(rows[0], pos_sh)

        pltpu.sync_copy(posidx_hbm, posidx_v)
        pltpu.sync_copy(idx_hbm.at[pl.ds(idx_base, IDX_ALL)], idx_v)
        plsc.subcore_barrier()

        def start_g(c, b):
            for j in range(IDX_ROWS):
                pltpu.async_copy(tok_hbm.at[idx_v.at[c * IDX_ROWS + j]],
                                 rows[b].at[pl.ds(j * IDX_W, IDX_W)], gsem[b])

        def wait_g(b):
            for j in range(IDX_ROWS):
                pltpu.make_async_copy(tok_hbm.at[idx_v.at[j]],
                                      rows[b].at[pl.ds(j * IDX_W, IDX_W)],
                                      gsem[b]).wait()

        def start_p(b):
            for j in range(IDX_ROWS):
                pltpu.async_copy(pos_sh.at[posidx_v.at[j]],
                                 rows[b].at[pl.ds(j * IDX_W, IDX_W)], psem[b],
                                 add=True)

        def wait_p(b):
            for j in range(IDX_ROWS):
                pltpu.make_async_copy(pos_sh.at[posidx_v.at[j]],
                                      rows[b].at[pl.ds(j * IDX_W, IDX_W)],
                                      psem[b]).wait()

        def start_s(c, b):
            pltpu.async_copy(rows[b],
                             out_hbm.at[pl.ds(row_base + c * CHUNK, CHUNK)],
                             ssem[b])

        def wait_s(b):
            pltpu.make_async_copy(rows[b],
                                  out_hbm.at[pl.ds(row_base, CHUNK)],
                                  ssem[b]).wait()

        for c in range(3):
            start_g(c, c)
        wait_g(0)
        start_p(0)

        @pl.loop(0, NUM_CHUNKS, step=NBUF)
        def _(cc):
            for b in range(NBUF):
                c = cc + b
                b1 = (b + 1) % NBUF

                @pl.when(c + 1 < NUM_CHUNKS)
                def _():
                    wait_g(b1)
                    start_p(b1)

                wait_p(b)
                start_s(c, b)

                b3 = (b + 3) % NBUF

                @pl.when(c + 3 < NUM_CHUNKS)
                def _():
                    @pl.when(c >= 1)
                    def _():
                        wait_s(b3)

                    start_g(c + 3, b3)

        for b in range(NBUF):
            wait_s(b)

    out = embed(idx2d, posidx, pos_table, tok_table)
    return out.reshape(BATCH, SEQ_LEN, EMBED_DIM)
